# baseline (speedup 1.0000x reference)
import functools

import jax
import jax.numpy as jnp
from jax import lax
from jax.experimental import pallas as pl
from jax.experimental.pallas import tpu as pltpu
from jax.experimental.pallas import tpu_sc as plsc

_K = 2
_TILE = 256
_RT = 512

_SC_CORES = 2
_SC_SUBCORES = 16
_SC_WORKERS = _SC_CORES * _SC_SUBCORES


def _router_body(x_ref, wr_ref, br_ref, gate_ref, idx_ref):
    l = jnp.dot(x_ref[...], wr_ref[...], preferred_element_type=jnp.float32)
    l = l + br_ref[...]
    m = jnp.max(l, axis=-1, keepdims=True)
    ex = jnp.exp(l - m)
    p = ex / jnp.sum(ex, axis=-1, keepdims=True)
    ncol = p.shape[-1]
    iota = lax.broadcasted_iota(jnp.int32, p.shape, 1)
    big = jnp.int32(ncol)
    m1 = jnp.max(p, axis=-1, keepdims=True)
    e1 = jnp.min(jnp.where(p == m1, iota, big), axis=-1, keepdims=True)
    p2 = jnp.where(iota == e1, -jnp.inf, p)
    m2 = jnp.max(p2, axis=-1, keepdims=True)
    e2 = jnp.min(jnp.where(p2 == m2, iota, big), axis=-1, keepdims=True)
    gate_ref[...] = jnp.concatenate([m1, m2], axis=1)
    idx_ref[...] = jnp.concatenate([e1, e2], axis=1)


def _router(xf, Wr, br):
    n, d = xf.shape
    e = Wr.shape[1]
    grid = n // _RT
    return pl.pallas_call(
        _router_body,
        grid=(grid,),
        in_specs=[
            pl.BlockSpec((_RT, d), lambda i: (i, 0)),
            pl.BlockSpec((d, e), lambda i: (0, 0)),
            pl.BlockSpec((1, e), lambda i: (0, 0)),
        ],
        out_specs=[
            pl.BlockSpec((_RT, _K), lambda i: (i, 0)),
            pl.BlockSpec((_RT, _K), lambda i: (i, 0)),
        ],
        out_shape=[
            jax.ShapeDtypeStruct((n, _K), jnp.float32),
            jax.ShapeDtypeStruct((n, _K), jnp.int32),
        ],
    )(xf, Wr.astype(jnp.float32), br.reshape(1, e).astype(jnp.float32))


def _sc_gather(table, idx):
    b = idx.shape[0]
    d = table.shape[1]
    b_per_w = b // _SC_WORKERS
    ch = min(32, b_per_w)
    n_ch = b_per_w // ch
    mesh = plsc.VectorSubcoreMesh(
        core_axis_name="c", subcore_axis_name="s",
        num_cores=_SC_CORES, num_subcores=_SC_SUBCORES)

    @functools.partial(
        pl.kernel, mesh=mesh,
        out_type=jax.ShapeDtypeStruct((b, d), table.dtype),
        scratch_types=[
            pltpu.VMEM((b_per_w,), jnp.int32),
            pltpu.VMEM((ch, d), table.dtype),
            pltpu.VMEM((ch, d), table.dtype),
            pltpu.SemaphoreType.DMA,
            pltpu.SemaphoreType.DMA,
            pltpu.SemaphoreType.DMA,
            pltpu.SemaphoreType.DMA,
        ],
    )
    def k(table_hbm, idx_hbm, out_hbm, idx_all, rows0, rows1,
          g0, g1, o0, o1):
        wid = lax.axis_index("s") * _SC_CORES + lax.axis_index("c")
        base = wid * b_per_w
        pltpu.sync_copy(idx_hbm.at[pl.ds(base, b_per_w)], idx_all)
        rows = (rows0, rows1)
        gsem = (g0, g1)
        osem = (o0, o1)

        def start_gather(c):
            return pltpu.async_copy(
                table_hbm.at[idx_all.at[pl.ds(c * ch, ch)]],
                rows[c % 2], gsem[c % 2])

        gathers = [None] * n_ch
        outs = [None] * n_ch
        gathers[0] = start_gather(0)
        for c in range(n_ch):
            if c + 1 < n_ch:
                if c >= 1:
                    outs[c - 1].wait()
                gathers[c + 1] = start_gather(c + 1)
            gathers[c].wait()
            outs[c] = pltpu.async_copy(
                rows[c % 2], out_hbm.at[pl.ds(base + c * ch, ch)],
                osem[c % 2])
        outs[n_ch - 1].wait()
        if n_ch >= 2:
            outs[n_ch - 2].wait()

    return k(table, idx)


def _ffn_body(vt_ref, ve_ref, xs_ref, w1_ref, b1_ref, w2_ref, b2_ref,
              se_ref, sg_ref, out_ref):
    v = pl.program_id(0)
    h = jnp.dot(xs_ref[...], w1_ref[0], preferred_element_type=jnp.float32)
    h = jax.nn.gelu(h + b1_ref[0]).astype(jnp.bfloat16)
    o = jnp.dot(h, w2_ref[0], preferred_element_type=jnp.float32)
    o = (o + b2_ref[0]) * sg_ref[0]
    mask = se_ref[0] == ve_ref[v]
    out_ref[...] = jnp.where(mask, o, out_ref[...])


def _grouped_ffn(visit_tile, visit_expert, xs, W1, b1, W2, b2, se3, sg3):
    a, d = xs.shape
    e, _, dff = W1.shape
    nt = a // _TILE
    nv = nt + e - 1
    grid_spec = pltpu.PrefetchScalarGridSpec(
        num_scalar_prefetch=2,
        grid=(nv,),
        in_specs=[
            pl.BlockSpec((_TILE, d), lambda v, vt, ve: (vt[v], 0)),
            pl.BlockSpec((1, d, dff), lambda v, vt, ve: (ve[v], 0, 0)),
            pl.BlockSpec((1, 1, dff), lambda v, vt, ve: (ve[v], 0, 0)),
            pl.BlockSpec((1, dff, d), lambda v, vt, ve: (ve[v], 0, 0)),
            pl.BlockSpec((1, 1, d), lambda v, vt, ve: (ve[v], 0, 0)),
            pl.BlockSpec((1, _TILE, 1), lambda v, vt, ve: (vt[v], 0, 0)),
            pl.BlockSpec((1, _TILE, 1), lambda v, vt, ve: (vt[v], 0, 0)),
        ],
        out_specs=pl.BlockSpec((_TILE, d), lambda v, vt, ve: (vt[v], 0)),
    )
    return pl.pallas_call(
        _ffn_body,
        grid_spec=grid_spec,
        out_shape=jax.ShapeDtypeStruct((a, d), jnp.float32),
        compiler_params=pltpu.CompilerParams(
            dimension_semantics=("arbitrary",)),
    )(visit_tile, visit_expert, xs,
      W1.astype(jnp.bfloat16), b1.reshape(e, 1, dff).astype(jnp.float32),
      W2.astype(jnp.bfloat16), b2.reshape(e, 1, d).astype(jnp.float32),
      se3, sg3)


def _ln_body(x_ref, y0_ref, y1_ref, g_ref, b_ref, o_ref):
    s = y0_ref[...] + y1_ref[...]
    mu = jnp.mean(s, axis=-1, keepdims=True)
    var = jnp.mean((s - mu) ** 2, axis=-1, keepdims=True)
    normed = (s - mu) / jnp.sqrt(var + 1e-5) * g_ref[...] + b_ref[...]
    o_ref[...] = x_ref[...] + normed


def _ln_residual(xf, ys01, gamma, beta):
    n, d = xf.shape
    grid = n // _RT
    half = n // _RT
    return pl.pallas_call(
        _ln_body,
        grid=(grid,),
        in_specs=[
            pl.BlockSpec((_RT, d), lambda i: (i, 0)),
            pl.BlockSpec((_RT, d), lambda i: (i, 0)),
            pl.BlockSpec((_RT, d), lambda i, h=half: (i + h, 0)),
            pl.BlockSpec((1, d), lambda i: (0, 0)),
            pl.BlockSpec((1, d), lambda i: (0, 0)),
        ],
        out_specs=pl.BlockSpec((_RT, d), lambda i: (i, 0)),
        out_shape=jax.ShapeDtypeStruct((n, d), jnp.float32),
    )(xf, ys01, ys01,
      gamma.reshape(1, d).astype(jnp.float32),
      beta.reshape(1, d).astype(jnp.float32))


def kernel(x, W1, b1, W2, b2, Wr, br, gamma, beta):
    bq, sq, d = x.shape
    e, _, dff = W1.shape
    n = bq * sq
    a = n * _K
    nt = a // _TILE
    xf = x.reshape(n, d).astype(jnp.float32)

    gate_top, idx_top = _router(xf, Wr, br)

    e_flat = idx_top.reshape(-1)
    g_flat = gate_top.reshape(-1)
    tok_flat = (jnp.arange(a, dtype=jnp.int32) // _K).astype(jnp.int32)
    perm = jnp.argsort(e_flat).astype(jnp.int32)
    sorted_e = e_flat[perm]
    sorted_tok = tok_flat[perm]
    sorted_g = g_flat[perm]
    inv = jnp.zeros((a,), jnp.int32).at[perm].set(
        jnp.arange(a, dtype=jnp.int32))
    pos01 = jnp.concatenate([inv[0::_K], inv[1::_K]])

    counts = jnp.bincount(e_flat, length=e).astype(jnp.int32)
    off = jnp.concatenate(
        [jnp.zeros((1,), jnp.int32), jnp.cumsum(counts).astype(jnp.int32)])
    t_lo = jnp.arange(nt, dtype=jnp.int32) * _TILE
    cnt_te = (jnp.minimum(off[1:][None, :], (t_lo + _TILE)[:, None])
              - jnp.maximum(off[:-1][None, :], t_lo[:, None]))
    present = (cnt_te > 0).reshape(-1)
    nv = nt + e - 1
    sentinel = jnp.int32(nt * e)
    keyv = jnp.where(present, jnp.arange(nt * e, dtype=jnp.int32), sentinel)
    skey = jnp.sort(keyv)[:nv]
    valid = skey < sentinel
    lastv = jnp.max(jnp.where(valid, skey, -1))
    skey = jnp.where(valid, skey, lastv)
    visit_tile = (skey // e).astype(jnp.int32)
    visit_expert = (skey % e).astype(jnp.int32)

    se3 = sorted_e.reshape(nt, _TILE, 1)
    sg3 = sorted_g.reshape(nt, _TILE, 1)

    xf_packed = jax.lax.bitcast_convert_type(
        xf.astype(jnp.bfloat16).reshape(n, d // 2, 2), jnp.int32)
    xs_packed = _sc_gather(xf_packed, sorted_tok)
    xs = jax.lax.bitcast_convert_type(
        xs_packed, jnp.bfloat16).reshape(a, d)
    ys = _grouped_ffn(visit_tile, visit_expert, xs, W1, b1, W2, b2, se3, sg3)
    ys01 = _sc_gather(ys, pos01)
    out_flat = _ln_residual(xf, ys01, gamma, beta)
    return out_flat.reshape(bq, sq, d)


# speedup vs baseline: 1.6373x; 1.6373x over previous
import functools

import jax
import jax.numpy as jnp
from jax import lax
from jax.experimental import pallas as pl
from jax.experimental.pallas import tpu as pltpu
from jax.experimental.pallas import tpu_sc as plsc

_K = 2
_TILE = 128
_RT = 512

_SC_CORES = 2
_SC_SUBCORES = 16
_SC_WORKERS = _SC_CORES * _SC_SUBCORES


def _router_body(x_ref, wr_ref, br_ref, gate_ref, idx_ref):
    l = jnp.dot(x_ref[...], wr_ref[...], preferred_element_type=jnp.float32)
    l = l + br_ref[...]
    m = jnp.max(l, axis=-1, keepdims=True)
    ex = jnp.exp(l - m)
    p = ex / jnp.sum(ex, axis=-1, keepdims=True)
    ncol = p.shape[-1]
    iota = lax.broadcasted_iota(jnp.int32, p.shape, 1)
    big = jnp.int32(ncol)
    m1 = jnp.max(p, axis=-1, keepdims=True)
    e1 = jnp.min(jnp.where(p == m1, iota, big), axis=-1, keepdims=True)
    p2 = jnp.where(iota == e1, -jnp.inf, p)
    m2 = jnp.max(p2, axis=-1, keepdims=True)
    e2 = jnp.min(jnp.where(p2 == m2, iota, big), axis=-1, keepdims=True)
    gate_ref[...] = jnp.concatenate([m1, m2], axis=1)
    idx_ref[...] = jnp.concatenate([e1, e2], axis=1)


def _router(xf, Wr, br):
    n, d = xf.shape
    e = Wr.shape[1]
    grid = n // _RT
    return pl.pallas_call(
        _router_body,
        grid=(grid,),
        in_specs=[
            pl.BlockSpec((_RT, d), lambda i: (i, 0)),
            pl.BlockSpec((d, e), lambda i: (0, 0)),
            pl.BlockSpec((1, e), lambda i: (0, 0)),
        ],
        out_specs=[
            pl.BlockSpec((_RT, _K), lambda i: (i, 0)),
            pl.BlockSpec((_RT, _K), lambda i: (i, 0)),
        ],
        out_shape=[
            jax.ShapeDtypeStruct((n, _K), jnp.float32),
            jax.ShapeDtypeStruct((n, _K), jnp.int32),
        ],
    )(xf, Wr.astype(jnp.float32), br.reshape(1, e).astype(jnp.float32))


def _sc_gather(table, idx):
    b = idx.shape[0]
    d = table.shape[1]
    b_per_w = b // _SC_WORKERS
    ch = min(32, b_per_w)
    n_ch = b_per_w // ch
    mesh = plsc.VectorSubcoreMesh(
        core_axis_name="c", subcore_axis_name="s",
        num_cores=_SC_CORES, num_subcores=_SC_SUBCORES)

    @functools.partial(
        pl.kernel, mesh=mesh,
        out_type=jax.ShapeDtypeStruct((b, d), jnp.float32),
        scratch_types=[
            pltpu.VMEM((b_per_w,), jnp.int32),
            pltpu.VMEM((ch, d), jnp.float32),
            pltpu.VMEM((ch, d), jnp.float32),
            pltpu.SemaphoreType.DMA,
            pltpu.SemaphoreType.DMA,
            pltpu.SemaphoreType.DMA,
            pltpu.SemaphoreType.DMA,
        ],
    )
    def k(table_hbm, idx_hbm, out_hbm, idx_all, rows0, rows1,
          g0, g1, o0, o1):
        wid = lax.axis_index("s") * _SC_CORES + lax.axis_index("c")
        base = wid * b_per_w
        pltpu.sync_copy(idx_hbm.at[pl.ds(base, b_per_w)], idx_all)
        rows = (rows0, rows1)
        gsem = (g0, g1)
        osem = (o0, o1)

        def start_gather(c):
            return pltpu.async_copy(
                table_hbm.at[idx_all.at[pl.ds(c * ch, ch)]],
                rows[c % 2], gsem[c % 2])

        gathers = [None] * n_ch
        outs = [None] * n_ch
        gathers[0] = start_gather(0)
        for c in range(n_ch):
            if c + 1 < n_ch:
                if c >= 1:
                    outs[c - 1].wait()
                gathers[c + 1] = start_gather(c + 1)
            gathers[c].wait()
            outs[c] = pltpu.async_copy(
                rows[c % 2], out_hbm.at[pl.ds(base + c * ch, ch)],
                osem[c % 2])
        outs[n_ch - 1].wait()
        if n_ch >= 2:
            outs[n_ch - 2].wait()

    return k(table, idx)


def _ffn_body(vt_ref, ve_ref, xs_ref, w1_ref, b1_ref, w2_ref, b2_ref,
              se_ref, sg_ref, out_ref):
    v = pl.program_id(0)
    h = jnp.dot(xs_ref[...], w1_ref[0], preferred_element_type=jnp.float32)
    h = jax.nn.gelu(h + b1_ref[0])
    o = jnp.dot(h, w2_ref[0], preferred_element_type=jnp.float32)
    o = (o + b2_ref[0]) * sg_ref[0]
    mask = se_ref[0] == ve_ref[v]
    out_ref[...] = jnp.where(mask, o, out_ref[...])


def _grouped_ffn(visit_tile, visit_expert, xs, W1, b1, W2, b2, se3, sg3):
    a, d = xs.shape
    e, _, dff = W1.shape
    nt = a // _TILE
    nv = nt + e - 1
    w_mode = pl.Buffered(buffer_count=1)
    grid_spec = pltpu.PrefetchScalarGridSpec(
        num_scalar_prefetch=2,
        grid=(nv,),
        in_specs=[
            pl.BlockSpec((_TILE, d), lambda v, vt, ve: (vt[v], 0)),
            pl.BlockSpec((1, d, dff), lambda v, vt, ve: (ve[v], 0, 0)),
            pl.BlockSpec((1, 1, dff), lambda v, vt, ve: (ve[v], 0, 0)),
            pl.BlockSpec((1, dff, d), lambda v, vt, ve: (ve[v], 0, 0),
                         pipeline_mode=w_mode),
            pl.BlockSpec((1, 1, d), lambda v, vt, ve: (ve[v], 0, 0)),
            pl.BlockSpec((1, _TILE, 1), lambda v, vt, ve: (vt[v], 0, 0)),
            pl.BlockSpec((1, _TILE, 1), lambda v, vt, ve: (vt[v], 0, 0)),
        ],
        out_specs=pl.BlockSpec((_TILE, d), lambda v, vt, ve: (vt[v], 0)),
    )
    return pl.pallas_call(
        _ffn_body,
        grid_spec=grid_spec,
        out_shape=jax.ShapeDtypeStruct((a, d), jnp.float32),
        compiler_params=pltpu.CompilerParams(
            dimension_semantics=("arbitrary",)),
    )(visit_tile, visit_expert, xs,
      W1.astype(jnp.float32), b1.reshape(e, 1, dff).astype(jnp.float32),
      W2.astype(jnp.float32), b2.reshape(e, 1, d).astype(jnp.float32),
      se3, sg3)


def _ln_body(x_ref, y0_ref, y1_ref, g_ref, b_ref, o_ref):
    s = y0_ref[...] + y1_ref[...]
    mu = jnp.mean(s, axis=-1, keepdims=True)
    var = jnp.mean((s - mu) ** 2, axis=-1, keepdims=True)
    normed = (s - mu) / jnp.sqrt(var + 1e-5) * g_ref[...] + b_ref[...]
    o_ref[...] = x_ref[...] + normed


def _ln_residual(xf, ys01, gamma, beta):
    n, d = xf.shape
    grid = n // _RT
    half = n // _RT
    return pl.pallas_call(
        _ln_body,
        grid=(grid,),
        in_specs=[
            pl.BlockSpec((_RT, d), lambda i: (i, 0)),
            pl.BlockSpec((_RT, d), lambda i: (i, 0)),
            pl.BlockSpec((_RT, d), lambda i, h=half: (i + h, 0)),
            pl.BlockSpec((1, d), lambda i: (0, 0)),
            pl.BlockSpec((1, d), lambda i: (0, 0)),
        ],
        out_specs=pl.BlockSpec((_RT, d), lambda i: (i, 0)),
        out_shape=jax.ShapeDtypeStruct((n, d), jnp.float32),
    )(xf, ys01, ys01,
      gamma.reshape(1, d).astype(jnp.float32),
      beta.reshape(1, d).astype(jnp.float32))


def kernel(x, W1, b1, W2, b2, Wr, br, gamma, beta):
    bq, sq, d = x.shape
    e, _, dff = W1.shape
    n = bq * sq
    a = n * _K
    nt = a // _TILE
    xf = x.reshape(n, d).astype(jnp.float32)

    gate_top, idx_top = _router(xf, Wr, br)

    e_flat = idx_top.reshape(-1)
    g_flat = gate_top.reshape(-1)
    tok_flat = (jnp.arange(a, dtype=jnp.int32) // _K).astype(jnp.int32)
    perm = jnp.argsort(e_flat).astype(jnp.int32)
    sorted_e = e_flat[perm]
    sorted_tok = tok_flat[perm]
    sorted_g = g_flat[perm]
    inv = jnp.zeros((a,), jnp.int32).at[perm].set(
        jnp.arange(a, dtype=jnp.int32))
    pos01 = jnp.concatenate([inv[0::_K], inv[1::_K]])

    counts = jnp.bincount(e_flat, length=e).astype(jnp.int32)
    off = jnp.concatenate(
        [jnp.zeros((1,), jnp.int32), jnp.cumsum(counts).astype(jnp.int32)])
    t_lo = jnp.arange(nt, dtype=jnp.int32) * _TILE
    cnt_te = (jnp.minimum(off[1:][None, :], (t_lo + _TILE)[:, None])
              - jnp.maximum(off[:-1][None, :], t_lo[:, None]))
    present = (cnt_te > 0).reshape(-1)
    nv = nt + e - 1
    sentinel = jnp.int32(nt * e)
    keyv = jnp.where(present, jnp.arange(nt * e, dtype=jnp.int32), sentinel)
    skey = jnp.sort(keyv)[:nv]
    valid = skey < sentinel
    lastv = jnp.max(jnp.where(valid, skey, -1))
    skey = jnp.where(valid, skey, lastv)
    visit_tile = (skey // e).astype(jnp.int32)
    visit_expert = (skey % e).astype(jnp.int32)

    se3 = sorted_e.reshape(nt, _TILE, 1)
    sg3 = sorted_g.reshape(nt, _TILE, 1)

    xs = _sc_gather(xf, sorted_tok)
    ys = _grouped_ffn(visit_tile, visit_expert, xs, W1, b1, W2, b2, se3, sg3)
    ys01 = _sc_gather(ys, pos01)
    out_flat = _ln_residual(xf, ys01, gamma, beta)
    return out_flat.reshape(bq, sq, d)


# speedup vs baseline: 1.7881x; 1.0921x over previous
import functools

import jax
import jax.numpy as jnp
from jax import lax
from jax.experimental import pallas as pl
from jax.experimental.pallas import tpu as pltpu
from jax.experimental.pallas import tpu_sc as plsc

_K = 2
_TILE = 256
_RT = 512

_SC_CORES = 2
_SC_SUBCORES = 16
_SC_WORKERS = _SC_CORES * _SC_SUBCORES


def _router_body(x_ref, wr_ref, br_ref, gate_ref, idx_ref):
    l = jnp.dot(x_ref[...], wr_ref[...], preferred_element_type=jnp.float32)
    l = l + br_ref[...]
    m = jnp.max(l, axis=-1, keepdims=True)
    ex = jnp.exp(l - m)
    p = ex / jnp.sum(ex, axis=-1, keepdims=True)
    ncol = p.shape[-1]
    iota = lax.broadcasted_iota(jnp.int32, p.shape, 1)
    big = jnp.int32(ncol)
    m1 = jnp.max(p, axis=-1, keepdims=True)
    e1 = jnp.min(jnp.where(p == m1, iota, big), axis=-1, keepdims=True)
    p2 = jnp.where(iota == e1, -jnp.inf, p)
    m2 = jnp.max(p2, axis=-1, keepdims=True)
    e2 = jnp.min(jnp.where(p2 == m2, iota, big), axis=-1, keepdims=True)
    gate_ref[...] = jnp.concatenate([m1, m2], axis=1)
    idx_ref[...] = jnp.concatenate([e1, e2], axis=1)


def _router(xf, Wr, br):
    n, d = xf.shape
    e = Wr.shape[1]
    grid = n // _RT
    return pl.pallas_call(
        _router_body,
        grid=(grid,),
        in_specs=[
            pl.BlockSpec((_RT, d), lambda i: (i, 0)),
            pl.BlockSpec((d, e), lambda i: (0, 0)),
            pl.BlockSpec((1, e), lambda i: (0, 0)),
        ],
        out_specs=[
            pl.BlockSpec((_RT, _K), lambda i: (i, 0)),
            pl.BlockSpec((_RT, _K), lambda i: (i, 0)),
        ],
        out_shape=[
            jax.ShapeDtypeStruct((n, _K), jnp.float32),
            jax.ShapeDtypeStruct((n, _K), jnp.int32),
        ],
    )(xf, Wr.astype(jnp.float32), br.reshape(1, e).astype(jnp.float32))


def _sc_gather(table, idx):
    b = idx.shape[0]
    d = table.shape[1]
    b_per_w = b // _SC_WORKERS
    ch = min(32, b_per_w)
    n_ch = b_per_w // ch
    mesh = plsc.VectorSubcoreMesh(
        core_axis_name="c", subcore_axis_name="s",
        num_cores=_SC_CORES, num_subcores=_SC_SUBCORES)

    @functools.partial(
        pl.kernel, mesh=mesh,
        out_type=jax.ShapeDtypeStruct((b, d), jnp.float32),
        scratch_types=[
            pltpu.VMEM((b_per_w,), jnp.int32),
            pltpu.VMEM((ch, d), jnp.float32),
            pltpu.VMEM((ch, d), jnp.float32),
            pltpu.SemaphoreType.DMA,
            pltpu.SemaphoreType.DMA,
            pltpu.SemaphoreType.DMA,
            pltpu.SemaphoreType.DMA,
        ],
    )
    def k(table_hbm, idx_hbm, out_hbm, idx_all, rows0, rows1,
          g0, g1, o0, o1):
        wid = lax.axis_index("s") * _SC_CORES + lax.axis_index("c")
        base = wid * b_per_w
        pltpu.sync_copy(idx_hbm.at[pl.ds(base, b_per_w)], idx_all)
        rows = (rows0, rows1)
        gsem = (g0, g1)
        osem = (o0, o1)

        def start_gather(c):
            return pltpu.async_copy(
                table_hbm.at[idx_all.at[pl.ds(c * ch, ch)]],
                rows[c % 2], gsem[c % 2])

        gathers = [None] * n_ch
        outs = [None] * n_ch
        gathers[0] = start_gather(0)
        for c in range(n_ch):
            if c + 1 < n_ch:
                if c >= 1:
                    outs[c - 1].wait()
                gathers[c + 1] = start_gather(c + 1)
            gathers[c].wait()
            outs[c] = pltpu.async_copy(
                rows[c % 2], out_hbm.at[pl.ds(base + c * ch, ch)],
                osem[c % 2])
        outs[n_ch - 1].wait()
        if n_ch >= 2:
            outs[n_ch - 2].wait()

    return k(table, idx)


def _ffn_body(vt_ref, ve_ref, vlo_ref, vhi_ref, xs_ref, w1_ref, b1_ref,
              w2_ref, b2_ref, out_ref):
    v = pl.program_id(0)
    h = jnp.dot(xs_ref[...], w1_ref[0], preferred_element_type=jnp.float32)
    h = jax.nn.gelu(h + b1_ref[0])
    o = jnp.dot(h, w2_ref[0], preferred_element_type=jnp.float32)
    o = o + b2_ref[0]
    row = lax.broadcasted_iota(jnp.int32, (o.shape[0], 1), 0)
    mask = (row >= vlo_ref[v]) & (row < vhi_ref[v])
    out_ref[...] = jnp.where(mask, o, out_ref[...])


def _grouped_ffn(visit_tile, visit_expert, visit_lo, visit_hi,
                 xs, W1, b1, W2, b2):
    a, d = xs.shape
    e, _, dff = W1.shape
    nt = a // _TILE
    nv = nt + e - 1
    w_mode = pl.Buffered(buffer_count=1)
    grid_spec = pltpu.PrefetchScalarGridSpec(
        num_scalar_prefetch=4,
        grid=(nv,),
        in_specs=[
            pl.BlockSpec((_TILE, d), lambda v, vt, ve, vlo, vhi: (vt[v], 0)),
            pl.BlockSpec((1, d, dff),
                         lambda v, vt, ve, vlo, vhi: (ve[v], 0, 0)),
            pl.BlockSpec((1, 1, dff),
                         lambda v, vt, ve, vlo, vhi: (ve[v], 0, 0)),
            pl.BlockSpec((1, dff, d),
                         lambda v, vt, ve, vlo, vhi: (ve[v], 0, 0),
                         pipeline_mode=w_mode),
            pl.BlockSpec((1, 1, d),
                         lambda v, vt, ve, vlo, vhi: (ve[v], 0, 0)),
        ],
        out_specs=pl.BlockSpec((_TILE, d),
                               lambda v, vt, ve, vlo, vhi: (vt[v], 0)),
    )
    return pl.pallas_call(
        _ffn_body,
        grid_spec=grid_spec,
        out_shape=jax.ShapeDtypeStruct((a, d), jnp.float32),
        compiler_params=pltpu.CompilerParams(
            dimension_semantics=("arbitrary",)),
    )(visit_tile, visit_expert, visit_lo, visit_hi, xs,
      W1.astype(jnp.float32), b1.reshape(e, 1, dff).astype(jnp.float32),
      W2.astype(jnp.float32), b2.reshape(e, 1, d).astype(jnp.float32))


def _ln_body(x_ref, y0_ref, y1_ref, g0_ref, g1_ref, g_ref, b_ref, o_ref):
    s = y0_ref[...] * g0_ref[...] + y1_ref[...] * g1_ref[...]
    mu = jnp.mean(s, axis=-1, keepdims=True)
    var = jnp.mean((s - mu) ** 2, axis=-1, keepdims=True)
    normed = (s - mu) / jnp.sqrt(var + 1e-5) * g_ref[...] + b_ref[...]
    o_ref[...] = x_ref[...] + normed


def _ln_residual(xf, ys01, gate_top, gamma, beta):
    n, d = xf.shape
    grid = n // _RT
    half = n // _RT
    return pl.pallas_call(
        _ln_body,
        grid=(grid,),
        in_specs=[
            pl.BlockSpec((_RT, d), lambda i: (i, 0)),
            pl.BlockSpec((_RT, d), lambda i: (i, 0)),
            pl.BlockSpec((_RT, d), lambda i, h=half: (i + h, 0)),
            pl.BlockSpec((_RT, 1), lambda i: (i, 0)),
            pl.BlockSpec((_RT, 1), lambda i: (i, 0)),
            pl.BlockSpec((1, d), lambda i: (0, 0)),
            pl.BlockSpec((1, d), lambda i: (0, 0)),
        ],
        out_specs=pl.BlockSpec((_RT, d), lambda i: (i, 0)),
        out_shape=jax.ShapeDtypeStruct((n, d), jnp.float32),
    )(xf, ys01, ys01, gate_top[:, 0:1], gate_top[:, 1:2],
      gamma.reshape(1, d).astype(jnp.float32),
      beta.reshape(1, d).astype(jnp.float32))


def kernel(x, W1, b1, W2, b2, Wr, br, gamma, beta):
    bq, sq, d = x.shape
    e, _, dff = W1.shape
    n = bq * sq
    a = n * _K
    nt = a // _TILE
    xf = x.reshape(n, d).astype(jnp.float32)

    gate_top, idx_top = _router(xf, Wr, br)

    e_flat = idx_top.reshape(-1)
    perm = jnp.argsort(e_flat).astype(jnp.int32)
    sorted_tok = perm // _K
    inv = jnp.zeros((a,), jnp.int32).at[perm].set(
        jnp.arange(a, dtype=jnp.int32))
    pos01 = jnp.concatenate([inv[0::_K], inv[1::_K]])

    counts = jnp.bincount(e_flat, length=e).astype(jnp.int32)
    off = jnp.concatenate(
        [jnp.zeros((1,), jnp.int32), jnp.cumsum(counts).astype(jnp.int32)])
    t_lo = jnp.arange(nt, dtype=jnp.int32) * _TILE
    lo_mat = jnp.clip(off[:-1][None, :] - t_lo[:, None], 0, _TILE)
    hi_mat = jnp.clip(off[1:][None, :] - t_lo[:, None], 0, _TILE)
    present = (hi_mat > lo_mat).reshape(-1)
    nv = nt + e - 1
    sentinel = jnp.int32(nt * e)
    keyv = jnp.where(present, jnp.arange(nt * e, dtype=jnp.int32), sentinel)
    skey = jnp.sort(keyv)[:nv]
    valid = skey < sentinel
    lastv = jnp.max(jnp.where(valid, skey, -1))
    skey = jnp.where(valid, skey, lastv)
    visit_tile = (skey // e).astype(jnp.int32)
    visit_expert = (skey % e).astype(jnp.int32)
    visit_lo = lo_mat.reshape(-1)[skey].astype(jnp.int32)
    visit_hi = hi_mat.reshape(-1)[skey].astype(jnp.int32)

    xs = _sc_gather(xf, sorted_tok)
    ys = _grouped_ffn(visit_tile, visit_expert, visit_lo, visit_hi,
                      xs, W1, b1, W2, b2)
    ys01 = _sc_gather(ys, pos01)
    out_flat = _ln_residual(xf, ys01, gate_top, gamma, beta)
    return out_flat.reshape(bq, sq, d)
